# Initial kernel scaffold; baseline (speedup 1.0000x reference)
import jax
import jax.numpy as jnp
from jax.experimental import pallas as pl


def kernel(center_words, context_negatives, center_table, context_table):
    raise NotImplementedError("write your pallas kernel here")


# speedup vs baseline: 1.5302x; 1.5302x over previous
import functools

import jax
import jax.numpy as jnp
from jax import lax
from jax.experimental import pallas as pl
from jax.experimental.pallas import tpu as pltpu
from jax.experimental.pallas import tpu_sc as plsc

V = 100000
D = 128
B = 1024
L = 200
N = B * L

NC = 2
NS = 16
NW = NC * NS
ROWS_PER_W = N // NW
CHUNK = 128
NCHUNK = ROWS_PER_W // CHUNK
IDX_ROWS = N // CHUNK


def _gather_body(cw_hbm, cn_hbm, ctab_hbm, xtab_hbm, vc_hbm, uc_hbm,
                 idx_v, rows_v, sem):
    wid = lax.axis_index("s") * NC + lax.axis_index("c")
    base = wid * ROWS_PER_W

    def one_table(idx_hbm, tab_hbm, out_hbm):
        pltpu.sync_copy(idx_hbm.at[pl.ds(base, ROWS_PER_W)], idx_v)

        def chunk(j, carry):
            idx_slice = idx_v.at[pl.ds(j * CHUNK, CHUNK)]
            pltpu.async_copy(tab_hbm.at[idx_slice], rows_v, sem).wait()
            pltpu.sync_copy(rows_v,
                            out_hbm.at[pl.ds(base + j * CHUNK, CHUNK)])
            return carry

        lax.fori_loop(0, NCHUNK, chunk, 0)

    one_table(cw_hbm, ctab_hbm, vc_hbm)
    one_table(cn_hbm, xtab_hbm, uc_hbm)


_gather = pl.kernel(
    _gather_body,
    out_type=(
        jax.ShapeDtypeStruct((N, D), jnp.float32),
        jax.ShapeDtypeStruct((N, D), jnp.float32),
    ),
    mesh=plsc.VectorSubcoreMesh(core_axis_name="c", subcore_axis_name="s"),
    scratch_types=[
        pltpu.VMEM((ROWS_PER_W,), jnp.int32),
        pltpu.VMEM((CHUNK, D), jnp.float32),
        pltpu.SemaphoreType.DMA,
    ],
)


def _bmm_body(v_ref, u_ref, o_ref):
    v = v_ref[0]
    u = u_ref[0]
    o_ref[0] = lax.dot_general(v, u, (((1,), (1,)), ((), ())),
                               preferred_element_type=jnp.float32)


_bmm = pl.pallas_call(
    _bmm_body,
    grid=(B,),
    in_specs=[
        pl.BlockSpec((1, L, D), lambda b: (b, 0, 0)),
        pl.BlockSpec((1, L, D), lambda b: (b, 0, 0)),
    ],
    out_specs=pl.BlockSpec((1, L, L), lambda b: (b, 0, 0)),
    out_shape=jax.ShapeDtypeStruct((B, L, L), jnp.float32),
)


def kernel(center_words, context_negatives, center_table, context_table):
    cw = center_words.reshape(N)
    cn = context_negatives.reshape(N)
    vc, uc = _gather(cw, cn, center_table, context_table)
    return _bmm(vc.reshape(B, L, D), uc.reshape(B, L, D))


# speedup vs baseline: 2.7801x; 1.8168x over previous
import functools

import jax
import jax.numpy as jnp
from jax import lax
from jax.experimental import pallas as pl
from jax.experimental.pallas import tpu as pltpu
from jax.experimental.pallas import tpu_sc as plsc

V = 100000
D = 128
B = 1024
L = 200
N = B * L

NC = 2
NS = 16
NW = NC * NS
ROWS_PER_W = N // NW
CHUNK = 128
NCHUNK = ROWS_PER_W // CHUNK
IDX_ROWS = N // CHUNK


def _gather_body(cw_hbm, cn_hbm, ctab_hbm, xtab_hbm, vc_hbm, uc_hbm,
                 idx_v, rows_v, sem):
    wid = lax.axis_index("s") * NC + lax.axis_index("c")
    base = wid * ROWS_PER_W

    def one_table(idx_hbm, tab_hbm, out_hbm):
        pltpu.sync_copy(idx_hbm.at[pl.ds(base, ROWS_PER_W)], idx_v)

        def chunk(j, carry):
            idx_slice = idx_v.at[pl.ds(j * CHUNK, CHUNK)]
            pltpu.async_copy(tab_hbm.at[idx_slice], rows_v, sem).wait()
            pltpu.sync_copy(rows_v,
                            out_hbm.at[pl.ds(base + j * CHUNK, CHUNK)])
            return carry

        lax.fori_loop(0, NCHUNK, chunk, 0)

    one_table(cw_hbm, ctab_hbm, vc_hbm)
    one_table(cn_hbm, xtab_hbm, uc_hbm)


_gather = pl.kernel(
    _gather_body,
    out_type=(
        jax.ShapeDtypeStruct((N, D), jnp.float32),
        jax.ShapeDtypeStruct((N, D), jnp.float32),
    ),
    mesh=plsc.VectorSubcoreMesh(core_axis_name="c", subcore_axis_name="s"),
    scratch_types=[
        pltpu.VMEM((ROWS_PER_W,), jnp.int32),
        pltpu.VMEM((CHUNK, D), jnp.float32),
        pltpu.SemaphoreType.DMA,
    ],
)


BG = 8


def _bmm_body(v_ref, u_ref, o_ref):
    for i in range(BG):
        o_ref[i] = lax.dot_general(v_ref[i], u_ref[i], (((1,), (1,)), ((), ())),
                                   preferred_element_type=jnp.float32)


_bmm = pl.pallas_call(
    _bmm_body,
    grid=(B // BG,),
    in_specs=[
        pl.BlockSpec((BG, L, D), lambda b: (b, 0, 0)),
        pl.BlockSpec((BG, L, D), lambda b: (b, 0, 0)),
    ],
    out_specs=pl.BlockSpec((BG, L, L), lambda b: (b, 0, 0)),
    out_shape=jax.ShapeDtypeStruct((B, L, L), jnp.float32),
)


def kernel(center_words, context_negatives, center_table, context_table):
    cw = center_words.reshape(N)
    cn = context_negatives.reshape(N)
    vc, uc = _gather(cw, cn, center_table, context_table)
    return _bmm(vc.reshape(B, L, D), uc.reshape(B, L, D))


# speedup vs baseline: 2.9806x; 1.0721x over previous
import jax
import jax.numpy as jnp
from jax import lax
from jax.experimental import pallas as pl
from jax.experimental.pallas import tpu as pltpu
from jax.experimental.pallas import tpu_sc as plsc

V = 100000
D = 128
B = 1024
L = 200
N = B * L

NC = 2
NS = 16
NW = NC * NS
ROWS_PER_W = N // NW
CHUNK = 256
NCHUNK = ROWS_PER_W // CHUNK


def _gather_body(cw_hbm, cn_hbm, ctab_hbm, xtab_hbm, vc_hbm, uc_hbm,
                 idx_v, rows_v, sem):
    wid = lax.axis_index("s") * NC + lax.axis_index("c")
    base = wid * ROWS_PER_W

    def one_table(idx_hbm, tab_hbm, out_hbm):
        pltpu.sync_copy(idx_hbm.at[pl.ds(base, ROWS_PER_W)], idx_v)

        def chunk(j, carry):
            idx_slice = idx_v.at[pl.ds(j * CHUNK, CHUNK)]
            pltpu.async_copy(tab_hbm.at[idx_slice], rows_v, sem).wait()
            pltpu.sync_copy(rows_v,
                            out_hbm.at[pl.ds(base + j * CHUNK, CHUNK)])
            return carry

        lax.fori_loop(0, NCHUNK, chunk, 0)

    one_table(cw_hbm, ctab_hbm, vc_hbm)
    one_table(cn_hbm, xtab_hbm, uc_hbm)


_gather = pl.kernel(
    _gather_body,
    out_type=(
        jax.ShapeDtypeStruct((N, D), jnp.float32),
        jax.ShapeDtypeStruct((N, D), jnp.float32),
    ),
    mesh=plsc.VectorSubcoreMesh(core_axis_name="c", subcore_axis_name="s"),
    scratch_types=[
        pltpu.VMEM((ROWS_PER_W,), jnp.int32),
        pltpu.VMEM((CHUNK, D), jnp.float32),
        pltpu.SemaphoreType.DMA,
    ],
)

BG = 8


def _bmm_body(v_ref, u_ref, o_ref):
    for i in range(BG):
        v = v_ref[i].astype(jnp.bfloat16)
        u = u_ref[i].astype(jnp.bfloat16)
        o_ref[i] = lax.dot_general(v, u, (((1,), (1,)), ((), ())),
                                   preferred_element_type=jnp.float32)


_bmm = pl.pallas_call(
    _bmm_body,
    grid=(B // BG,),
    in_specs=[
        pl.BlockSpec((BG, L, D), lambda b: (b, 0, 0)),
        pl.BlockSpec((BG, L, D), lambda b: (b, 0, 0)),
    ],
    out_specs=pl.BlockSpec((BG, L, L), lambda b: (b, 0, 0)),
    out_shape=jax.ShapeDtypeStruct((B, L, L), jnp.float32),
)


def kernel(center_words, context_negatives, center_table, context_table):
    cw = center_words.reshape(N)
    cn = context_negatives.reshape(N)
    vc, uc = _gather(cw, cn, center_table, context_table)
    return _bmm(vc.reshape(B, L, D), uc.reshape(B, L, D))


# speedup vs baseline: 3.1907x; 1.0705x over previous
import jax
import jax.numpy as jnp
from jax import lax
from jax.experimental import pallas as pl
from jax.experimental.pallas import tpu as pltpu
from jax.experimental.pallas import tpu_sc as plsc

V = 100000
D = 128
B = 1024
L = 200
N = B * L

NC = 2
NS = 16
NW = NC * NS
ROWS_PER_W = N // NW
CHUNK = 256
NCHUNK = ROWS_PER_W // CHUNK


def _gather_body(cw_hbm, cn_hbm, ctab_hbm, xtab_hbm, vc_hbm, uc_hbm,
                 idx_v, rows_v, sem):
    wid = lax.axis_index("s") * NC + lax.axis_index("c")
    base = wid * ROWS_PER_W

    def one_table(idx_hbm, tab_hbm, out_hbm):
        pltpu.sync_copy(idx_hbm.at[pl.ds(base, ROWS_PER_W)], idx_v)

        def chunk(j, carry):
            idx_slice = idx_v.at[pl.ds(j * CHUNK, CHUNK)]
            pltpu.async_copy(tab_hbm.at[idx_slice], rows_v, sem).wait()
            pltpu.sync_copy(rows_v,
                            out_hbm.at[pl.ds(base + j * CHUNK, CHUNK)])
            return carry

        lax.fori_loop(0, NCHUNK, chunk, 0)

    one_table(cw_hbm, ctab_hbm, vc_hbm)
    one_table(cn_hbm, xtab_hbm, uc_hbm)


_gather = pl.kernel(
    _gather_body,
    out_type=(
        jax.ShapeDtypeStruct((N, D), jnp.float32),
        jax.ShapeDtypeStruct((N, D), jnp.float32),
    ),
    mesh=plsc.VectorSubcoreMesh(core_axis_name="c", subcore_axis_name="s"),
    scratch_types=[
        pltpu.VMEM((ROWS_PER_W,), jnp.int32),
        pltpu.VMEM((CHUNK, D), jnp.float32),
        pltpu.SemaphoreType.DMA,
    ],
)

BG = 16


def _bmm_body(v_ref, u_ref, o_ref):
    for i in range(BG):
        v = v_ref[i].astype(jnp.bfloat16)
        u = u_ref[i].astype(jnp.bfloat16)
        o_ref[i] = lax.dot_general(v, u, (((1,), (1,)), ((), ())),
                                   preferred_element_type=jnp.float32)


_bmm = pl.pallas_call(
    _bmm_body,
    grid=(B // BG,),
    in_specs=[
        pl.BlockSpec((BG, L, D), lambda b: (b, 0, 0)),
        pl.BlockSpec((BG, L, D), lambda b: (b, 0, 0)),
    ],
    out_specs=pl.BlockSpec((BG, L, L), lambda b: (b, 0, 0)),
    out_shape=jax.ShapeDtypeStruct((B, L, L), jnp.float32),
)


def kernel(center_words, context_negatives, center_table, context_table):
    cw = center_words.reshape(N)
    cn = context_negatives.reshape(N)
    vc, uc = _gather(cw, cn, center_table, context_table)
    return _bmm(vc.reshape(B, L, D), uc.reshape(B, L, D))


# speedup vs baseline: 3.2059x; 1.0047x over previous
import jax
import jax.numpy as jnp
from jax import lax
from jax.experimental import pallas as pl
from jax.experimental.pallas import tpu as pltpu
from jax.experimental.pallas import tpu_sc as plsc

V = 100000
D = 128
B = 1024
L = 200
N = B * L

NC = 2
NS = 16
NW = NC * NS
ROWS_PER_W = N // NW
CHUNK = 256
NCHUNK = ROWS_PER_W // CHUNK


def _gather_body(cw_hbm, cn_hbm, ctab_hbm, xtab_hbm, vc_hbm, uc_hbm,
                 idx_v, rows_v, sem):
    wid = lax.axis_index("s") * NC + lax.axis_index("c")
    base = wid * ROWS_PER_W

    def one_table(idx_hbm, tab_hbm, out_hbm):
        pltpu.sync_copy(idx_hbm.at[pl.ds(base, ROWS_PER_W)], idx_v)

        def chunk(j, carry):
            idx_slice = idx_v.at[pl.ds(j * CHUNK, CHUNK)]
            pltpu.async_copy(tab_hbm.at[idx_slice], rows_v, sem).wait()
            pltpu.sync_copy(rows_v,
                            out_hbm.at[pl.ds(base + j * CHUNK, CHUNK)])
            return carry

        lax.fori_loop(0, NCHUNK, chunk, 0)

    one_table(cw_hbm, ctab_hbm, vc_hbm)
    one_table(cn_hbm, xtab_hbm, uc_hbm)


_gather = pl.kernel(
    _gather_body,
    out_type=(
        jax.ShapeDtypeStruct((N, D), jnp.float32),
        jax.ShapeDtypeStruct((N, D), jnp.float32),
    ),
    mesh=plsc.VectorSubcoreMesh(core_axis_name="c", subcore_axis_name="s"),
    scratch_types=[
        pltpu.VMEM((ROWS_PER_W,), jnp.int32),
        pltpu.VMEM((CHUNK, D), jnp.float32),
        pltpu.SemaphoreType.DMA,
    ],
)

BG = 32


def _bmm_body(v_ref, u_ref, o_ref):
    for i in range(BG):
        v = v_ref[i].astype(jnp.bfloat16)
        u = u_ref[i].astype(jnp.bfloat16)
        o_ref[i] = lax.dot_general(v, u, (((1,), (1,)), ((), ())),
                                   preferred_element_type=jnp.float32)


_bmm = pl.pallas_call(
    _bmm_body,
    grid=(B // BG,),
    in_specs=[
        pl.BlockSpec((BG, L, D), lambda b: (b, 0, 0)),
        pl.BlockSpec((BG, L, D), lambda b: (b, 0, 0)),
    ],
    out_specs=pl.BlockSpec((BG, L, L), lambda b: (b, 0, 0)),
    out_shape=jax.ShapeDtypeStruct((B, L, L), jnp.float32),
)


def kernel(center_words, context_negatives, center_table, context_table):
    cw = center_words.reshape(N)
    cn = context_negatives.reshape(N)
    vc, uc = _gather(cw, cn, center_table, context_table)
    return _bmm(vc.reshape(B, L, D), uc.reshape(B, L, D))


# speedup vs baseline: 3.2083x; 1.0008x over previous
import jax
import jax.numpy as jnp
from jax import lax
from jax.experimental import pallas as pl
from jax.experimental.pallas import tpu as pltpu
from jax.experimental.pallas import tpu_sc as plsc

V = 100000
D = 128
B = 1024
L = 200
N = B * L

NCHK = 2
BH = B // NCHK
NH = BH * L

NC = 2
NS = 16
NW = NC * NS
ROWS_PER_W = NH // NW
CHUNK = 320
NCHUNK = ROWS_PER_W // CHUNK


def _gather_body(cw_hbm, cn_hbm, ctab_hbm, xtab_hbm, vc_hbm, uc_hbm,
                 idx_v, rows_v, sem):
    wid = lax.axis_index("s") * NC + lax.axis_index("c")
    base = wid * ROWS_PER_W

    def one_table(idx_hbm, tab_hbm, out_hbm):
        pltpu.sync_copy(idx_hbm.at[pl.ds(base, ROWS_PER_W)], idx_v)

        def chunk(j, carry):
            idx_slice = idx_v.at[pl.ds(j * CHUNK, CHUNK)]
            pltpu.async_copy(tab_hbm.at[idx_slice], rows_v, sem).wait()
            pltpu.sync_copy(rows_v,
                            out_hbm.at[pl.ds(base + j * CHUNK, CHUNK)])
            return carry

        lax.fori_loop(0, NCHUNK, chunk, 0)

    one_table(cw_hbm, ctab_hbm, vc_hbm)
    one_table(cn_hbm, xtab_hbm, uc_hbm)


_gather = pl.kernel(
    _gather_body,
    out_type=(
        jax.ShapeDtypeStruct((NH, D), jnp.float32),
        jax.ShapeDtypeStruct((NH, D), jnp.float32),
    ),
    mesh=plsc.VectorSubcoreMesh(core_axis_name="c", subcore_axis_name="s"),
    scratch_types=[
        pltpu.VMEM((ROWS_PER_W,), jnp.int32),
        pltpu.VMEM((CHUNK, D), jnp.float32),
        pltpu.SemaphoreType.DMA,
    ],
)

BG = 32
HSTEPS = BH // BG


def _bmm_body(v0_ref, u0_ref, v1_ref, u1_ref, o_ref):
    b = pl.program_id(0)

    def emit(v_ref, u_ref):
        for i in range(BG):
            v = v_ref[i].astype(jnp.bfloat16)
            u = u_ref[i].astype(jnp.bfloat16)
            o_ref[i] = lax.dot_general(v, u, (((1,), (1,)), ((), ())),
                                       preferred_element_type=jnp.float32)

    @pl.when(b < HSTEPS)
    def _():
        emit(v0_ref, u0_ref)

    @pl.when(b >= HSTEPS)
    def _():
        emit(v1_ref, u1_ref)


def _lo(b):
    return (jnp.minimum(b, HSTEPS - 1), 0, 0)


def _hi(b):
    return (jnp.maximum(b - HSTEPS, 0), 0, 0)


_bmm = pl.pallas_call(
    _bmm_body,
    grid=(B // BG,),
    in_specs=[
        pl.BlockSpec((BG, L, D), _lo),
        pl.BlockSpec((BG, L, D), _lo),
        pl.BlockSpec((BG, L, D), _hi),
        pl.BlockSpec((BG, L, D), _hi),
    ],
    out_specs=pl.BlockSpec((BG, L, L), lambda b: (b, 0, 0)),
    out_shape=jax.ShapeDtypeStruct((B, L, L), jnp.float32),
)


def kernel(center_words, context_negatives, center_table, context_table):
    cw = center_words.reshape(NCHK, NH)
    cn = context_negatives.reshape(NCHK, NH)
    vc0, uc0 = _gather(cw[0], cn[0], center_table, context_table)
    vc1, uc1 = _gather(cw[1], cn[1], center_table, context_table)
    return _bmm(vc0.reshape(BH, L, D), uc0.reshape(BH, L, D),
                vc1.reshape(BH, L, D), uc1.reshape(BH, L, D))


# speedup vs baseline: 3.3240x; 1.0360x over previous
import jax
import jax.numpy as jnp
from jax import lax
from jax.experimental import pallas as pl
from jax.experimental.pallas import tpu as pltpu
from jax.experimental.pallas import tpu_sc as plsc

V = 100000
D = 128
B = 1024
L = 200
N = B * L

NCHK = 4
BH = B // NCHK
NH = BH * L

NC = 2
NS = 16
NW = NC * NS
ROWS_PER_W = NH // NW
CHUNK = 320
NCHUNK = ROWS_PER_W // CHUNK


def _gather_body(cw_hbm, cn_hbm, ctab_hbm, xtab_hbm, vc_hbm, uc_hbm,
                 idx_v, rows_v, sem):
    wid = lax.axis_index("s") * NC + lax.axis_index("c")
    base = wid * ROWS_PER_W

    def one_table(idx_hbm, tab_hbm, out_hbm):
        pltpu.sync_copy(idx_hbm.at[pl.ds(base, ROWS_PER_W)], idx_v)

        def chunk(j, carry):
            idx_slice = idx_v.at[pl.ds(j * CHUNK, CHUNK)]
            pltpu.async_copy(tab_hbm.at[idx_slice], rows_v, sem).wait()
            pltpu.sync_copy(rows_v,
                            out_hbm.at[pl.ds(base + j * CHUNK, CHUNK)])
            return carry

        lax.fori_loop(0, NCHUNK, chunk, 0)

    one_table(cw_hbm, ctab_hbm, vc_hbm)
    one_table(cn_hbm, xtab_hbm, uc_hbm)


_gather = pl.kernel(
    _gather_body,
    out_type=(
        jax.ShapeDtypeStruct((NH, D), jnp.float32),
        jax.ShapeDtypeStruct((NH, D), jnp.float32),
    ),
    mesh=plsc.VectorSubcoreMesh(core_axis_name="c", subcore_axis_name="s"),
    scratch_types=[
        pltpu.VMEM((ROWS_PER_W,), jnp.int32),
        pltpu.VMEM((CHUNK, D), jnp.float32),
        pltpu.SemaphoreType.DMA,
    ],
)

BG = 32
HSTEPS = BH // BG


def _bmm_first_body(v_ref, u_ref, o_ref):
    for i in range(BG):
        v = v_ref[i].astype(jnp.bfloat16)
        u = u_ref[i].astype(jnp.bfloat16)
        o_ref[i] = lax.dot_general(v, u, (((1,), (1,)), ((), ())),
                                   preferred_element_type=jnp.float32)


def _bmm_chunk_body(full_ref, v_ref, u_ref, o_ref):
    _bmm_first_body(v_ref, u_ref, o_ref)


def _bmm_first(vc, uc):
    return pl.pallas_call(
        _bmm_first_body,
        grid=(HSTEPS,),
        in_specs=[
            pl.BlockSpec((BG, L, D), lambda b: (b, 0, 0)),
            pl.BlockSpec((BG, L, D), lambda b: (b, 0, 0)),
        ],
        out_specs=pl.BlockSpec((BG, L, L), lambda b: (b, 0, 0)),
        out_shape=jax.ShapeDtypeStruct((B, L, L), jnp.float32),
    )(vc, uc)


def _bmm_chunk(full, vc, uc, c):
    return pl.pallas_call(
        _bmm_chunk_body,
        grid=(HSTEPS,),
        in_specs=[
            pl.BlockSpec(memory_space=pl.ANY),
            pl.BlockSpec((BG, L, D), lambda b: (b, 0, 0)),
            pl.BlockSpec((BG, L, D), lambda b: (b, 0, 0)),
        ],
        out_specs=pl.BlockSpec((BG, L, L), lambda b, c=c: (b + c * HSTEPS, 0, 0)),
        out_shape=jax.ShapeDtypeStruct((B, L, L), jnp.float32),
        input_output_aliases={0: 0},
    )(full, vc, uc)


def kernel(center_words, context_negatives, center_table, context_table):
    cw = center_words.reshape(NCHK, NH)
    cn = context_negatives.reshape(NCHK, NH)
    gathered = [_gather(cw[c], cn[c], center_table, context_table)
                for c in range(NCHK)]
    full = _bmm_first(gathered[0][0].reshape(BH, L, D),
                      gathered[0][1].reshape(BH, L, D))
    for c in range(1, NCHK):
        full = _bmm_chunk(full,
                          gathered[c][0].reshape(BH, L, D),
                          gathered[c][1].reshape(BH, L, D), c)
    return full
